# baseline (speedup 1.0000x reference)
import functools

import jax
import jax.numpy as jnp
from jax import lax
from jax.experimental import pallas as pl
from jax.experimental.pallas import tpu as pltpu
from jax.experimental.pallas import tpu_sc as plsc

NE = 50000
NU = 20000
NI = 5
EMB = 64
HALF = 32
NREL = 23
NEDGE = 800000
NNZ = 1000000

CHUNK = 128
GRP = 1
EPAD = 802816
NNZPAD = 1003520
ECHUNKS_PER_TILE = 392
UCHUNKS_PER_TILE = 490
NE_ACC = 50008
CPROWS = 500

_f32 = jnp.float32
_bf16 = jnp.bfloat16
_i32 = jnp.int32

_MESH = dict(core_axis_name="c", subcore_axis_name="s")


def _perm_halves(x):
    n = x.shape[0]
    return x.reshape(n, 2, 16, 2).swapaxes(2, 3).reshape(2 * n, 32)


def _pack_bf16(pa, pb):
    return plsc.pack(pa, pb, format=plsc.PackFormat.INTERLEAVED)


_SC_PARAMS = pltpu.CompilerParams(
    use_tc_tiling_on_sc=False, needs_layout_passes=False)


def _sc_cnt(headp):
    @functools.partial(
        pl.kernel,
        out_type=jax.ShapeDtypeStruct((2, NE, 16), _f32),
        mesh=plsc.VectorSubcoreMesh(**_MESH),
        compiler_params=_SC_PARAMS,
        scratch_types=[
            pltpu.VMEM((CHUNK, 16), _f32),
            pltpu.VMEM((CPROWS, 16), _f32),
            pltpu.VMEM((GRP, 128), _i32),
            pltpu.VMEM_SHARED((NE_ACC, 16), _f32),
            pltpu.SemaphoreType.DMA,
        ],
    )
    def k(headp_h, out_h, onesb, zb, raw0, cacc, sem):
        c = lax.axis_index("c")
        s = lax.axis_index("s")
        onev = jnp.ones((16,), _f32)
        zv = jnp.zeros((16,), _f32)

        def fill_body(i, _):
            onesb[i, pl.ds(0, 16)] = onev
            return 0

        lax.fori_loop(0, CHUNK, fill_body, 0, unroll=8)

        def z_body(i, _):
            zb[i, pl.ds(0, 16)] = zv
            return 0

        lax.fori_loop(0, CPROWS, z_body, 0, unroll=8)

        def zc_body(z, _):
            r = s + z * 16

            @pl.when(r < 100)
            def _():
                pltpu.sync_copy(zb, cacc.at[pl.ds(r * CPROWS, CPROWS), :])
            return 0

        lax.fori_loop(0, 7, zc_body, 0)
        plsc.subcore_barrier()

        def chunk_body(i, _):
            j = (c * 16 + s) * 196 + i
            pltpu.sync_copy(headp_h.at[pl.ds(j * GRP, GRP), :], raw0)
            for g in range(GRP):
                pltpu.sync_copy(
                    onesb.at[pl.ds(g * 128, 128), :],
                    cacc.at[raw0.at[g]],
                    add=True,
                )
            return 0

        lax.fori_loop(0, 196, chunk_body, 0)
        plsc.subcore_barrier()

        def cp_body(z, _):
            r = s + z * 16

            @pl.when(r < 100)
            def _():
                pltpu.sync_copy(cacc.at[pl.ds(r * CPROWS, CPROWS), :], zb)
                pltpu.sync_copy(zb, out_h.at[c, pl.ds(r * CPROWS, CPROWS), :])
            return 0

        lax.fori_loop(0, 7, cp_body, 0)

    return k(headp)


def _sc_hop(ent2, r2, headp, tailp, typep, rowp, colp, valp):
    @functools.partial(
        pl.kernel,
        out_type=[
            jax.ShapeDtypeStruct((2, NE, HALF), _bf16),
            jax.ShapeDtypeStruct((2, NU, HALF), _bf16),
        ],
        mesh=plsc.VectorSubcoreMesh(**_MESH),
        compiler_params=_SC_PARAMS,
        scratch_types=[
            pltpu.VMEM((CHUNK, HALF), _f32),
            pltpu.VMEM((CHUNK, HALF), _f32),
            pltpu.VMEM((2 * NREL, HALF), _f32),
            pltpu.VMEM((CHUNK, HALF), _bf16),
            pltpu.VMEM((CHUNK, HALF), _bf16),
            pltpu.VMEM((GRP, 128), _i32),
            pltpu.VMEM((GRP, 128), _i32),
            pltpu.VMEM((GRP, 128), _i32),
            pltpu.VMEM((GRP, 128), _i32),
            pltpu.VMEM((GRP, 128), _i32),
            pltpu.VMEM((GRP, 128), _i32),
            pltpu.VMEM((GRP, 128), _f32),
            pltpu.VMEM((GRP, 128), _f32),
            pltpu.VMEM((GRP, 128), _i32),
            pltpu.VMEM((GRP, 128), _i32),
            pltpu.VMEM((GRP, 128), _i32),
            pltpu.VMEM((CPROWS, HALF), _bf16),
            pltpu.VMEM_SHARED((NE_ACC, HALF), _bf16),
            pltpu.SemaphoreType.DMA,
            pltpu.SemaphoreType.DMA,
            pltpu.SemaphoreType.DMA,
            pltpu.SemaphoreType.DMA,
            pltpu.SemaphoreType.DMA,
        ],
    )
    def k(ent2_h, r2_h, headp_h, tailp_h, typep_h, rowp_h, colp_h, valp_h,
          esum_h, usum_h,
          tA0, tA1, rtmp, tP0, tP1, gidx0, gidx1, relg0, relg1,
          sidx0, sidx1, valb0, valb1, raw0, raw1, raw2, zb, acc,
          sem, gsa, gsb, ssa, ssb):
        c = lax.axis_index("c")
        s = lax.axis_index("s")
        zv = jnp.zeros((32,), _bf16)
        A = dict(tA=tA0, tP=tP0, gidx=gidx0, relg=relg0, sidx=sidx0,
                 valb=valb0, gs=gsa, ss=ssa)
        B = dict(tA=tA1, tP=tP1, gidx=gidx1, relg=relg1, sidx=sidx1,
                 valb=valb1, gs=gsb, ss=ssb)
        pltpu.sync_copy(r2_h, rtmp)

        def z_body(i, _):
            zb[i, :] = zv
            return 0

        def zero_rows(nchunks, zmax):
            def zc_body(z, _):
                r = s + z * 16

                @pl.when(r < nchunks)
                def _():
                    pltpu.sync_copy(zb, acc.at[pl.ds(r * CPROWS, CPROWS), :])
                return 0

            lax.fori_loop(0, zmax, zc_body, 0)

        def copy_out(nchunks, zmax, out_ref):
            def cp_body(z, _):
                r = s + z * 16

                @pl.when(r < nchunks)
                def _():
                    pltpu.sync_copy(acc.at[pl.ds(r * CPROWS, CPROWS), :], zb)
                    pltpu.sync_copy(
                        zb, out_ref.at[c, pl.ds(r * CPROWS, CPROWS), :])
                return 0

            lax.fori_loop(0, zmax, cp_body, 0)

        lax.fori_loop(0, CPROWS, z_body, 0, unroll=8)
        zero_rows(100, 7)
        plsc.subcore_barrier()

        ebase = s * ECHUNKS_PER_TILE

        def eload_raws(n):
            base = (ebase + n) * GRP
            dl = [pltpu.async_copy(headp_h.at[pl.ds(base, GRP), :], raw0, sem),
                  pltpu.async_copy(tailp_h.at[pl.ds(base, GRP), :], raw1, sem),
                  pltpu.async_copy(typep_h.at[pl.ds(base, GRP), :], raw2, sem)]
            for d in dl:
                d.wait()

        def eidx(S):
            def idxb(q, _):
                g = q // 8
                off = (q % 8) * 16
                t = raw1[g, pl.ds(off, 16)]
                S["gidx"][g, pl.ds(off, 16)] = t + t + c
                ty = raw2[g, pl.ds(off, 16)] + 22
                m = ty - jnp.where(ty >= NREL, NREL, 0).astype(_i32)
                S["relg"][g, pl.ds(off, 16)] = m + m + c
                S["sidx"][g, pl.ds(off, 16)] = raw0[g, pl.ds(off, 16)]
                return 0

            lax.fori_loop(0, 8, idxb, 0, unroll=4)

        def efire_gathers(S):
            for g in range(GRP):
                pltpu.async_copy(ent2_h.at[S["gidx"].at[g]],
                                 S["tA"].at[pl.ds(g * 128, 128), :], S["gs"])

        def ewait_gathers(S):
            pltpu.make_async_copy(
                ent2_h.at[pl.ds(0, CHUNK)], S["tA"], S["gs"]).wait()

        def edrain_scatters(S):
            pltpu.make_async_copy(
                esum_h.at[c, pl.ds(0, CHUNK), :], S["tP"], S["ss"]).wait()

        dn0 = lax.GatherDimensionNumbers(
            offset_dims=(), collapsed_slice_dims=(0,), start_index_map=(0,))
        iota16 = jnp.arange(16, dtype=_i32)
        iota16p = iota16 + 16

        def emul(S):
            def mulb(q, _):
                g = q // 8
                off = (q % 8) * 16
                rr = S["relg"][g, pl.ds(off, 16)]
                for lane in range(16):
                    li = jnp.full((16, 1), lane, _i32)
                    rsp = lax.gather(
                        rr, li, dn0, slice_sizes=(1,),
                        mode=lax.GatherScatterMode.PROMISE_IN_BOUNDS)
                    va = plsc.load_gather(rtmp, [rsp, iota16])
                    vb = plsc.load_gather(rtmp, [rsp, iota16p])
                    k2 = q * 16 + lane
                    pa = S["tA"][k2, pl.ds(0, 16)] * va
                    pb = S["tA"][k2, pl.ds(16, 16)] * vb
                    S["tP"][k2, :] = _pack_bf16(pa, pb)
                return 0

            lax.fori_loop(0, 8, mulb, 0)

        def efire_scatters(S):
            for g in range(GRP):
                pltpu.async_copy(S["tP"].at[pl.ds(g * 128, 128), :],
                                 acc.at[S["sidx"].at[g]], S["ss"], add=True)

        eload_raws(0)
        eidx(A)
        efire_gathers(A)

        NP = ECHUNKS_PER_TILE // 2

        def epair(i, _):
            @pl.when(i > 0)
            def _():
                edrain_scatters(B)
            eload_raws(2 * i + 1)
            eidx(B)
            efire_gathers(B)
            ewait_gathers(A)
            emul(A)
            efire_scatters(A)
            @pl.when(i < NP - 1)
            def _():
                edrain_scatters(A)
                eload_raws(2 * i + 2)
                eidx(A)
                efire_gathers(A)
            ewait_gathers(B)
            emul(B)
            efire_scatters(B)
            return 0

        lax.fori_loop(0, NP, epair, 0)
        edrain_scatters(A)
        edrain_scatters(B)
        plsc.subcore_barrier()
        copy_out(100, 7, esum_h)
        plsc.subcore_barrier()

        lax.fori_loop(0, CPROWS, z_body, 0, unroll=8)
        zero_rows(40, 3)
        plsc.subcore_barrier()

        ubase = s * UCHUNKS_PER_TILE
        dn = lax.GatherDimensionNumbers(
            offset_dims=(), collapsed_slice_dims=(0,), start_index_map=(0,))

        def uload_raws(n, S):
            base = (ubase + n) * GRP
            dl = [pltpu.async_copy(rowp_h.at[pl.ds(base, GRP), :], raw0, sem),
                  pltpu.async_copy(colp_h.at[pl.ds(base, GRP), :], raw1, sem),
                  pltpu.async_copy(valp_h.at[pl.ds(base, GRP), :],
                                   S["valb"], sem)]
            for d in dl:
                d.wait()

        def uidx(S):
            def idxb(q, _):
                g = q // 8
                off = (q % 8) * 16
                t = raw1[g, pl.ds(off, 16)]
                S["gidx"][g, pl.ds(off, 16)] = t + t + c
                S["sidx"][g, pl.ds(off, 16)] = raw0[g, pl.ds(off, 16)]
                return 0

            lax.fori_loop(0, 8, idxb, 0, unroll=4)

        def ufire_gathers(S):
            for g in range(GRP):
                pltpu.async_copy(ent2_h.at[S["gidx"].at[g]],
                                 S["tA"].at[pl.ds(g * 128, 128), :], S["gs"])

        def uwait_gathers(S):
            pltpu.make_async_copy(
                ent2_h.at[pl.ds(0, CHUNK)], S["tA"], S["gs"]).wait()

        def umul(S):
            def vmulb(q, _):
                g = q // 8
                off = (q % 8) * 16
                vals16 = S["valb"][g, pl.ds(off, 16)]
                for lane in range(16):
                    li = jnp.full((16, 1), lane, _i32)
                    vs = lax.gather(
                        vals16, li, dn, slice_sizes=(1,),
                        mode=lax.GatherScatterMode.PROMISE_IN_BOUNDS)
                    k2 = q * 16 + lane
                    pa = S["tA"][k2, pl.ds(0, 16)] * vs
                    pb = S["tA"][k2, pl.ds(16, 16)] * vs
                    S["tP"][k2, :] = _pack_bf16(pa, pb)
                return 0

            lax.fori_loop(0, 8, vmulb, 0)

        uload_raws(0, A)
        uidx(A)
        ufire_gathers(A)

        NUP = UCHUNKS_PER_TILE // 2

        def upair(i, _):
            @pl.when(i > 0)
            def _():
                edrain_scatters(B)
            uload_raws(2 * i + 1, B)
            uidx(B)
            ufire_gathers(B)
            uwait_gathers(A)
            umul(A)
            efire_scatters(A)

            @pl.when(i < NUP - 1)
            def _():
                edrain_scatters(A)
                uload_raws(2 * i + 2, A)
                uidx(A)
                ufire_gathers(A)
            uwait_gathers(B)
            umul(B)
            efire_scatters(B)
            return 0

        lax.fori_loop(0, NUP, upair, 0)
        edrain_scatters(A)
        edrain_scatters(B)
        plsc.subcore_barrier()
        copy_out(40, 3, usum_h)

    return k(ent2, r2, headp, tailp, typep, rowp, colp, valp)


def _dcorr_centered(tr, tc):
    outer = tc * tr
    a = jnp.sqrt(jnp.maximum(tc * tc - 2.0 * outer + tr * tr, 0.0) + 1e-8)
    return (a - jnp.mean(a, axis=0, keepdims=True)
            - jnp.mean(a, axis=1, keepdims=True) + jnp.mean(a))


def _dcorr(t1r, t1c, t2r, t2c):
    n2 = float(EMB * EMB)
    A = _dcorr_centered(t1r, t1c)
    B = _dcorr_centered(t2r, t2c)
    dab = jnp.sqrt(jnp.maximum(jnp.sum(A * B) / n2, 0.0) + 1e-8)
    daa = jnp.sqrt(jnp.maximum(jnp.sum(A * A) / n2, 0.0) + 1e-8)
    dbb = jnp.sqrt(jnp.maximum(jnp.sum(B * B) / n2, 0.0) + 1e-8)
    return dab / jnp.sqrt(daa * dbb + 1e-8)


def _tc_intent_body(intent_ref, intentT_ref, r_ref, upd_ref, cor_ref):
    intent = intent_ref[...]
    intentT = intentT_ref[...]
    r = r_ref[...]

    def part(vec_row, emb):
        sc = jnp.sum(vec_row * emb, axis=1, keepdims=True)
        att = jax.nn.softmax(sc, axis=0)
        return jnp.mean(att * emb, axis=0, keepdims=True)

    parts = [part(intent[0:1], r),
             part(intent[1:2], r[0:6]),
             part(intent[2:3], r[6:12]),
             part(intent[3:4], r[12:18]),
             part(intent[4:5], r[18:23])]
    all_intent = jnp.concatenate(parts, axis=0)
    upd_ref[...] = (all_intent + intent) / 2.0
    cor = jnp.float32(0.0)
    for i in range(NI):
        for j in range(i + 1, NI):
            cor = cor + _dcorr(intent[i:i + 1, :], intentT[:, i:i + 1],
                               intent[j:j + 1, :], intentT[:, j:j + 1])
    cor_ref[...] = cor.reshape(1, 1)


def _tc_intent(intent_emb, intent_embT, r_emb):
    return pl.pallas_call(
        _tc_intent_body,
        out_shape=[jax.ShapeDtypeStruct((NI, EMB), _f32),
                   jax.ShapeDtypeStruct((1, 1), _f32)],
    )(intent_emb, intent_embT, r_emb)


def _tc_ent_body(esum_ref, cnt_ref, res_ref, e_ref, rout_ref):
    es = jnp.concatenate([esum_ref[0], esum_ref[1]], axis=1).astype(_f32)
    cnt = cnt_ref[0, :, 0:1] + cnt_ref[1, :, 0:1]
    x = es / jnp.maximum(cnt, 1.0)
    nrm = jnp.sqrt(jnp.sum(x * x, axis=1, keepdims=True))
    e = x / jnp.maximum(nrm, 1e-12)
    e_ref[...] = e
    rout_ref[...] = res_ref[...] + e


def _tc_entity(esum, cntp, res_in):
    blk = 2000
    return pl.pallas_call(
        _tc_ent_body,
        grid=(NE // blk,),
        in_specs=[
            pl.BlockSpec((2, blk, HALF), lambda i: (0, i, 0)),
            pl.BlockSpec((2, blk, 16), lambda i: (0, i, 0)),
            pl.BlockSpec((blk, EMB), lambda i: (i, 0)),
        ],
        out_specs=[
            pl.BlockSpec((blk, EMB), lambda i: (i, 0)),
            pl.BlockSpec((blk, EMB), lambda i: (i, 0)),
        ],
        out_shape=[jax.ShapeDtypeStruct((NE, EMB), _f32),
                   jax.ShapeDtypeStruct((NE, EMB), _f32)],
    )(esum, cntp, res_in)


def _tc_user_body(usum_ref, uprev_ref, upd_ref, res_ref, u_ref, rout_ref):
    us = jnp.concatenate([usum_ref[0], usum_ref[1]], axis=1).astype(_f32)
    up = upd_ref[...]
    score_ = lax.dot_general(uprev_ref[...], up, (((1,), (1,)), ((), ())),
                             preferred_element_type=_f32)
    score = jax.nn.softmax(score_, axis=1)
    f = lax.dot_general(score, up, (((1,), (0,)), ((), ())),
                        preferred_element_type=_f32)
    u = us * (1.0 + f)
    nrm = jnp.sqrt(jnp.sum(u * u, axis=1, keepdims=True))
    un = u / jnp.maximum(nrm, 1e-12)
    u_ref[...] = un
    rout_ref[...] = res_ref[...] + un


def _tc_user(usum, u_prev, intent_upd, res_in):
    blk = 2000
    return pl.pallas_call(
        _tc_user_body,
        grid=(NU // blk,),
        in_specs=[
            pl.BlockSpec((2, blk, HALF), lambda i: (0, i, 0)),
            pl.BlockSpec((blk, EMB), lambda i: (i, 0)),
            pl.BlockSpec((NI, EMB), lambda i: (0, 0)),
            pl.BlockSpec((blk, EMB), lambda i: (i, 0)),
        ],
        out_specs=[
            pl.BlockSpec((blk, EMB), lambda i: (i, 0)),
            pl.BlockSpec((blk, EMB), lambda i: (i, 0)),
        ],
        out_shape=[jax.ShapeDtypeStruct((NU, EMB), _f32),
                   jax.ShapeDtypeStruct((NU, EMB), _f32)],
    )(usum, u_prev, intent_upd, res_in)


def kernel(entity_emb, user_emb, intent_emb, edge_index, edge_type,
           interact_row, interact_col, interact_val, r_emb):
    head = edge_index[0].astype(_i32)
    tail = edge_index[1].astype(_i32)
    etype = edge_type.astype(_i32)
    row = interact_row.astype(_i32)
    col = interact_col.astype(_i32)
    val = interact_val.astype(_f32)

    headp = jnp.concatenate(
        [head, jnp.full((EPAD - NEDGE,), NE, _i32)]).reshape(-1, 128)
    tailp = jnp.concatenate(
        [tail, jnp.zeros((EPAD - NEDGE,), _i32)]).reshape(-1, 128)
    typep = jnp.concatenate(
        [etype, jnp.ones((EPAD - NEDGE,), _i32)]).reshape(-1, 128)
    rowp = jnp.concatenate(
        [row, jnp.full((NNZPAD - NNZ,), NU, _i32)]).reshape(-1, 128)
    colp = jnp.concatenate(
        [col, jnp.zeros((NNZPAD - NNZ,), _i32)]).reshape(-1, 128)
    valp = jnp.concatenate(
        [val, jnp.zeros((NNZPAD - NNZ,), _f32)]).reshape(-1, 128)
    r2 = _perm_halves(r_emb)

    cntp = _sc_cnt(headp)
    intent_upd, cor2 = _tc_intent(intent_emb, intent_emb.T, r_emb)

    def hop_body(carry, _):
        e, u, eres, ures = carry
        esum, usum = _sc_hop(_perm_halves(e), r2, headp, tailp, typep,
                             rowp, colp, valp)
        enew, eres2 = _tc_entity(esum, cntp, eres)
        unew, ures2 = _tc_user(usum, u, intent_upd, ures)
        return (enew, unew, eres2, ures2), None

    (_, _, eres, ures), _ = lax.scan(
        hop_body, (entity_emb, user_emb, entity_emb, user_emb), None, length=2)
    return eres, ures, cor2.reshape(())


# speedup vs baseline: 1.3814x; 1.3814x over previous
import functools

import jax
import jax.numpy as jnp
from jax import lax
from jax.experimental import pallas as pl
from jax.experimental.pallas import tpu as pltpu
from jax.experimental.pallas import tpu_sc as plsc

NE = 50000
NU = 20000
NI = 5
EMB = 64
HALF = 32
NREL = 23
NEDGE = 800000
NNZ = 1000000

CHUNK = 256
GRP = 2
EPAD = 802816
NNZPAD = 1007616
ECHUNKS_PER_TILE = 196
UCHUNKS_PER_TILE = 246
NE_ACC = 50008
CPROWS = 500

_f32 = jnp.float32
_bf16 = jnp.bfloat16
_i32 = jnp.int32

_MESH = dict(core_axis_name="c", subcore_axis_name="s")


def _perm_halves(x):
    n = x.shape[0]
    return x.reshape(n, 2, 16, 2).swapaxes(2, 3).reshape(2 * n, 32)


def _pack_bf16(pa, pb):
    return plsc.pack(pa, pb, format=plsc.PackFormat.INTERLEAVED)


_SC_PARAMS = pltpu.CompilerParams(
    use_tc_tiling_on_sc=False, needs_layout_passes=False)


def _sc_cnt(headp):
    @functools.partial(
        pl.kernel,
        out_type=jax.ShapeDtypeStruct((2, NE, 16), _f32),
        mesh=plsc.VectorSubcoreMesh(**_MESH),
        compiler_params=_SC_PARAMS,
        scratch_types=[
            pltpu.VMEM((CHUNK, 16), _f32),
            pltpu.VMEM((CPROWS, 16), _f32),
            pltpu.VMEM((GRP, 128), _i32),
            pltpu.VMEM_SHARED((NE_ACC, 16), _f32),
            pltpu.SemaphoreType.DMA,
        ],
    )
    def k(headp_h, out_h, onesb, zb, raw0, cacc, sem):
        c = lax.axis_index("c")
        s = lax.axis_index("s")
        onev = jnp.ones((16,), _f32)
        zv = jnp.zeros((16,), _f32)

        def fill_body(i, _):
            onesb[i, pl.ds(0, 16)] = onev
            return 0

        lax.fori_loop(0, CHUNK, fill_body, 0, unroll=8)

        def z_body(i, _):
            zb[i, pl.ds(0, 16)] = zv
            return 0

        lax.fori_loop(0, CPROWS, z_body, 0, unroll=8)

        def zc_body(z, _):
            r = s + z * 16

            @pl.when(r < 100)
            def _():
                pltpu.sync_copy(zb, cacc.at[pl.ds(r * CPROWS, CPROWS), :])
            return 0

        lax.fori_loop(0, 7, zc_body, 0)
        plsc.subcore_barrier()

        def chunk_body(i, _):
            j = (c * 16 + s) * 98 + i
            pltpu.sync_copy(headp_h.at[pl.ds(j * GRP, GRP), :], raw0)
            for g in range(GRP):
                pltpu.sync_copy(
                    onesb.at[pl.ds(g * 128, 128), :],
                    cacc.at[raw0.at[g]],
                    add=True,
                )
            return 0

        lax.fori_loop(0, 98, chunk_body, 0)
        plsc.subcore_barrier()

        def cp_body(z, _):
            r = s + z * 16

            @pl.when(r < 100)
            def _():
                pltpu.sync_copy(cacc.at[pl.ds(r * CPROWS, CPROWS), :], zb)
                pltpu.sync_copy(zb, out_h.at[c, pl.ds(r * CPROWS, CPROWS), :])
            return 0

        lax.fori_loop(0, 7, cp_body, 0)

    return k(headp)


def _sc_hop(ent2, r2, headp, tailp, typep, rowp, colp, valp):
    @functools.partial(
        pl.kernel,
        out_type=[
            jax.ShapeDtypeStruct((2, NE, HALF), _bf16),
            jax.ShapeDtypeStruct((2, NU, HALF), _bf16),
        ],
        mesh=plsc.VectorSubcoreMesh(**_MESH),
        compiler_params=_SC_PARAMS,
        scratch_types=[
            pltpu.VMEM((CHUNK, HALF), _bf16),
            pltpu.VMEM((CHUNK, HALF), _bf16),
            pltpu.VMEM((2 * NREL, HALF), _f32),
            pltpu.VMEM((CHUNK, HALF), _bf16),
            pltpu.VMEM((CHUNK, HALF), _bf16),
            pltpu.VMEM((GRP, 128), _i32),
            pltpu.VMEM((GRP, 128), _i32),
            pltpu.VMEM((GRP, 128), _i32),
            pltpu.VMEM((GRP, 128), _i32),
            pltpu.VMEM((GRP, 128), _i32),
            pltpu.VMEM((GRP, 128), _i32),
            pltpu.VMEM((GRP, 128), _f32),
            pltpu.VMEM((GRP, 128), _f32),
            pltpu.VMEM((GRP, 128), _i32),
            pltpu.VMEM((GRP, 128), _i32),
            pltpu.VMEM((GRP, 128), _i32),
            pltpu.VMEM((CPROWS, HALF), _bf16),
            pltpu.VMEM_SHARED((NE_ACC, HALF), _bf16),
            pltpu.SemaphoreType.DMA,
            pltpu.SemaphoreType.DMA,
            pltpu.SemaphoreType.DMA,
            pltpu.SemaphoreType.DMA,
            pltpu.SemaphoreType.DMA,
        ],
    )
    def k(ent2_h, r2_h, headp_h, tailp_h, typep_h, rowp_h, colp_h, valp_h,
          esum_h, usum_h,
          tA0, tA1, rtmp, tP0, tP1, gidx0, gidx1, relg0, relg1,
          sidx0, sidx1, valb0, valb1, raw0, raw1, raw2, zb, acc,
          sem, gsa, gsb, ssa, ssb):
        c = lax.axis_index("c")
        s = lax.axis_index("s")
        zv = jnp.zeros((32,), _bf16)
        A = dict(tA=tA0, tP=tP0, gidx=gidx0, relg=relg0, sidx=sidx0,
                 valb=valb0, gs=gsa, ss=ssa)
        B = dict(tA=tA1, tP=tP1, gidx=gidx1, relg=relg1, sidx=sidx1,
                 valb=valb1, gs=gsb, ss=ssb)
        pltpu.sync_copy(r2_h, rtmp)

        def z_body(i, _):
            zb[i, :] = zv
            return 0

        def zero_rows(nchunks, zmax):
            def zc_body(z, _):
                r = s + z * 16

                @pl.when(r < nchunks)
                def _():
                    pltpu.sync_copy(zb, acc.at[pl.ds(r * CPROWS, CPROWS), :])
                return 0

            lax.fori_loop(0, zmax, zc_body, 0)

        def copy_out(nchunks, zmax, out_ref):
            def cp_body(z, _):
                r = s + z * 16

                @pl.when(r < nchunks)
                def _():
                    pltpu.sync_copy(acc.at[pl.ds(r * CPROWS, CPROWS), :], zb)
                    pltpu.sync_copy(
                        zb, out_ref.at[c, pl.ds(r * CPROWS, CPROWS), :])
                return 0

            lax.fori_loop(0, zmax, cp_body, 0)

        lax.fori_loop(0, CPROWS, z_body, 0, unroll=8)
        zero_rows(100, 7)
        plsc.subcore_barrier()

        ebase = s * ECHUNKS_PER_TILE

        def eload_raws(n):
            base = (ebase + n) * GRP
            dl = [pltpu.async_copy(headp_h.at[pl.ds(base, GRP), :], raw0, sem),
                  pltpu.async_copy(tailp_h.at[pl.ds(base, GRP), :], raw1, sem),
                  pltpu.async_copy(typep_h.at[pl.ds(base, GRP), :], raw2, sem)]
            for d in dl:
                d.wait()

        def eidx(S):
            def idxb(q, _):
                g = q // 8
                off = (q % 8) * 16
                t = raw1[g, pl.ds(off, 16)]
                S["gidx"][g, pl.ds(off, 16)] = t + t + c
                ty = raw2[g, pl.ds(off, 16)] + 22
                m = ty - jnp.where(ty >= NREL, NREL, 0).astype(_i32)
                S["relg"][g, pl.ds(off, 16)] = m + m + c
                S["sidx"][g, pl.ds(off, 16)] = raw0[g, pl.ds(off, 16)]
                return 0

            lax.fori_loop(0, 16, idxb, 0, unroll=4)

        def efire_gathers(S):
            for g in range(GRP):
                pltpu.async_copy(ent2_h.at[S["gidx"].at[g]],
                                 S["tA"].at[pl.ds(g * 128, 128), :], S["gs"])

        def ewait_gathers(S):
            pltpu.make_async_copy(
                ent2_h.at[pl.ds(0, CHUNK)], S["tA"], S["gs"]).wait()

        def edrain_scatters(S):
            pltpu.make_async_copy(
                esum_h.at[c, pl.ds(0, CHUNK), :], S["tP"], S["ss"]).wait()

        dn0 = lax.GatherDimensionNumbers(
            offset_dims=(), collapsed_slice_dims=(0,), start_index_map=(0,))
        iota16 = jnp.arange(16, dtype=_i32)
        iota16p = iota16 + 16

        def emul(S):
            def mulb(q, _):
                g = q // 8
                off = (q % 8) * 16
                rr = S["relg"][g, pl.ds(off, 16)]
                for lane in range(16):
                    li = jnp.full((16, 1), lane, _i32)
                    rsp = lax.gather(
                        rr, li, dn0, slice_sizes=(1,),
                        mode=lax.GatherScatterMode.PROMISE_IN_BOUNDS)
                    va = plsc.load_gather(rtmp, [rsp, iota16])
                    vb = plsc.load_gather(rtmp, [rsp, iota16p])
                    k2 = q * 16 + lane
                    w = plsc.bitcast(S["tA"][k2, :], _i32)
                    ea = plsc.bitcast(lax.shift_left(w, 16), _f32)
                    eb = plsc.bitcast(w & jnp.int32(-65536), _f32)
                    S["tP"][k2, :] = _pack_bf16(ea * va, eb * vb)
                return 0

            lax.fori_loop(0, 16, mulb, 0)

        def efire_scatters(S):
            for g in range(GRP):
                pltpu.async_copy(S["tP"].at[pl.ds(g * 128, 128), :],
                                 acc.at[S["sidx"].at[g]], S["ss"], add=True)

        eload_raws(0)
        eidx(A)
        efire_gathers(A)

        NP = ECHUNKS_PER_TILE // 2

        def epair(i, _):
            @pl.when(i > 0)
            def _():
                edrain_scatters(B)
            eload_raws(2 * i + 1)
            eidx(B)
            efire_gathers(B)
            ewait_gathers(A)
            emul(A)
            efire_scatters(A)
            @pl.when(i < NP - 1)
            def _():
                edrain_scatters(A)
                eload_raws(2 * i + 2)
                eidx(A)
                efire_gathers(A)
            ewait_gathers(B)
            emul(B)
            efire_scatters(B)
            return 0

        lax.fori_loop(0, NP, epair, 0)
        edrain_scatters(A)
        edrain_scatters(B)
        plsc.subcore_barrier()
        copy_out(100, 7, esum_h)
        plsc.subcore_barrier()

        lax.fori_loop(0, CPROWS, z_body, 0, unroll=8)
        zero_rows(40, 3)
        plsc.subcore_barrier()

        ubase = s * UCHUNKS_PER_TILE
        dn = lax.GatherDimensionNumbers(
            offset_dims=(), collapsed_slice_dims=(0,), start_index_map=(0,))

        def uload_raws(n, S):
            base = (ubase + n) * GRP
            dl = [pltpu.async_copy(rowp_h.at[pl.ds(base, GRP), :], raw0, sem),
                  pltpu.async_copy(colp_h.at[pl.ds(base, GRP), :], raw1, sem),
                  pltpu.async_copy(valp_h.at[pl.ds(base, GRP), :],
                                   S["valb"], sem)]
            for d in dl:
                d.wait()

        def uidx(S):
            def idxb(q, _):
                g = q // 8
                off = (q % 8) * 16
                t = raw1[g, pl.ds(off, 16)]
                S["gidx"][g, pl.ds(off, 16)] = t + t + c
                S["sidx"][g, pl.ds(off, 16)] = raw0[g, pl.ds(off, 16)]
                return 0

            lax.fori_loop(0, 16, idxb, 0, unroll=4)

        def ufire_gathers(S):
            for g in range(GRP):
                pltpu.async_copy(ent2_h.at[S["gidx"].at[g]],
                                 S["tA"].at[pl.ds(g * 128, 128), :], S["gs"])

        def uwait_gathers(S):
            pltpu.make_async_copy(
                ent2_h.at[pl.ds(0, CHUNK)], S["tA"], S["gs"]).wait()

        def umul(S):
            def vmulb(q, _):
                g = q // 8
                off = (q % 8) * 16
                vals16 = S["valb"][g, pl.ds(off, 16)]
                for lane in range(16):
                    li = jnp.full((16, 1), lane, _i32)
                    vs = lax.gather(
                        vals16, li, dn, slice_sizes=(1,),
                        mode=lax.GatherScatterMode.PROMISE_IN_BOUNDS)
                    k2 = q * 16 + lane
                    w = plsc.bitcast(S["tA"][k2, :], _i32)
                    ea = plsc.bitcast(lax.shift_left(w, 16), _f32)
                    eb = plsc.bitcast(w & jnp.int32(-65536), _f32)
                    S["tP"][k2, :] = _pack_bf16(ea * vs, eb * vs)
                return 0

            lax.fori_loop(0, 16, vmulb, 0)

        uload_raws(0, A)
        uidx(A)
        ufire_gathers(A)

        NUP = UCHUNKS_PER_TILE // 2

        def upair(i, _):
            @pl.when(i > 0)
            def _():
                edrain_scatters(B)
            uload_raws(2 * i + 1, B)
            uidx(B)
            ufire_gathers(B)
            uwait_gathers(A)
            umul(A)
            efire_scatters(A)

            @pl.when(i < NUP - 1)
            def _():
                edrain_scatters(A)
                uload_raws(2 * i + 2, A)
                uidx(A)
                ufire_gathers(A)
            uwait_gathers(B)
            umul(B)
            efire_scatters(B)
            return 0

        lax.fori_loop(0, NUP, upair, 0)
        edrain_scatters(A)
        edrain_scatters(B)
        plsc.subcore_barrier()
        copy_out(40, 3, usum_h)

    return k(ent2, r2, headp, tailp, typep, rowp, colp, valp)


def _dcorr_centered(tr, tc):
    outer = tc * tr
    a = jnp.sqrt(jnp.maximum(tc * tc - 2.0 * outer + tr * tr, 0.0) + 1e-8)
    return (a - jnp.mean(a, axis=0, keepdims=True)
            - jnp.mean(a, axis=1, keepdims=True) + jnp.mean(a))


def _dcorr(t1r, t1c, t2r, t2c):
    n2 = float(EMB * EMB)
    A = _dcorr_centered(t1r, t1c)
    B = _dcorr_centered(t2r, t2c)
    dab = jnp.sqrt(jnp.maximum(jnp.sum(A * B) / n2, 0.0) + 1e-8)
    daa = jnp.sqrt(jnp.maximum(jnp.sum(A * A) / n2, 0.0) + 1e-8)
    dbb = jnp.sqrt(jnp.maximum(jnp.sum(B * B) / n2, 0.0) + 1e-8)
    return dab / jnp.sqrt(daa * dbb + 1e-8)


def _tc_intent_body(intent_ref, intentT_ref, r_ref, upd_ref, cor_ref):
    intent = intent_ref[...]
    intentT = intentT_ref[...]
    r = r_ref[...]

    def part(vec_row, emb):
        sc = jnp.sum(vec_row * emb, axis=1, keepdims=True)
        att = jax.nn.softmax(sc, axis=0)
        return jnp.mean(att * emb, axis=0, keepdims=True)

    parts = [part(intent[0:1], r),
             part(intent[1:2], r[0:6]),
             part(intent[2:3], r[6:12]),
             part(intent[3:4], r[12:18]),
             part(intent[4:5], r[18:23])]
    all_intent = jnp.concatenate(parts, axis=0)
    upd_ref[...] = (all_intent + intent) / 2.0
    cor = jnp.float32(0.0)
    for i in range(NI):
        for j in range(i + 1, NI):
            cor = cor + _dcorr(intent[i:i + 1, :], intentT[:, i:i + 1],
                               intent[j:j + 1, :], intentT[:, j:j + 1])
    cor_ref[...] = cor.reshape(1, 1)


def _tc_intent(intent_emb, intent_embT, r_emb):
    return pl.pallas_call(
        _tc_intent_body,
        out_shape=[jax.ShapeDtypeStruct((NI, EMB), _f32),
                   jax.ShapeDtypeStruct((1, 1), _f32)],
    )(intent_emb, intent_embT, r_emb)


def _tc_ent_body(esum_ref, cnt_ref, res_ref, e_ref, rout_ref):
    es = jnp.concatenate([esum_ref[0], esum_ref[1]], axis=1).astype(_f32)
    cnt = cnt_ref[0, :, 0:1] + cnt_ref[1, :, 0:1]
    x = es / jnp.maximum(cnt, 1.0)
    nrm = jnp.sqrt(jnp.sum(x * x, axis=1, keepdims=True))
    e = x / jnp.maximum(nrm, 1e-12)
    e_ref[...] = e
    rout_ref[...] = res_ref[...] + e


def _tc_entity(esum, cntp, res_in):
    blk = 2000
    return pl.pallas_call(
        _tc_ent_body,
        grid=(NE // blk,),
        in_specs=[
            pl.BlockSpec((2, blk, HALF), lambda i: (0, i, 0)),
            pl.BlockSpec((2, blk, 16), lambda i: (0, i, 0)),
            pl.BlockSpec((blk, EMB), lambda i: (i, 0)),
        ],
        out_specs=[
            pl.BlockSpec((blk, EMB), lambda i: (i, 0)),
            pl.BlockSpec((blk, EMB), lambda i: (i, 0)),
        ],
        out_shape=[jax.ShapeDtypeStruct((NE, EMB), _f32),
                   jax.ShapeDtypeStruct((NE, EMB), _f32)],
    )(esum, cntp, res_in)


def _tc_user_body(usum_ref, uprev_ref, upd_ref, res_ref, u_ref, rout_ref):
    us = jnp.concatenate([usum_ref[0], usum_ref[1]], axis=1).astype(_f32)
    up = upd_ref[...]
    score_ = lax.dot_general(uprev_ref[...], up, (((1,), (1,)), ((), ())),
                             preferred_element_type=_f32)
    score = jax.nn.softmax(score_, axis=1)
    f = lax.dot_general(score, up, (((1,), (0,)), ((), ())),
                        preferred_element_type=_f32)
    u = us * (1.0 + f)
    nrm = jnp.sqrt(jnp.sum(u * u, axis=1, keepdims=True))
    un = u / jnp.maximum(nrm, 1e-12)
    u_ref[...] = un
    rout_ref[...] = res_ref[...] + un


def _tc_user(usum, u_prev, intent_upd, res_in):
    blk = 2000
    return pl.pallas_call(
        _tc_user_body,
        grid=(NU // blk,),
        in_specs=[
            pl.BlockSpec((2, blk, HALF), lambda i: (0, i, 0)),
            pl.BlockSpec((blk, EMB), lambda i: (i, 0)),
            pl.BlockSpec((NI, EMB), lambda i: (0, 0)),
            pl.BlockSpec((blk, EMB), lambda i: (i, 0)),
        ],
        out_specs=[
            pl.BlockSpec((blk, EMB), lambda i: (i, 0)),
            pl.BlockSpec((blk, EMB), lambda i: (i, 0)),
        ],
        out_shape=[jax.ShapeDtypeStruct((NU, EMB), _f32),
                   jax.ShapeDtypeStruct((NU, EMB), _f32)],
    )(usum, u_prev, intent_upd, res_in)


def kernel(entity_emb, user_emb, intent_emb, edge_index, edge_type,
           interact_row, interact_col, interact_val, r_emb):
    head = edge_index[0].astype(_i32)
    tail = edge_index[1].astype(_i32)
    etype = edge_type.astype(_i32)
    row = interact_row.astype(_i32)
    col = interact_col.astype(_i32)
    val = interact_val.astype(_f32)

    headp = jnp.concatenate(
        [head, jnp.full((EPAD - NEDGE,), NE, _i32)]).reshape(-1, 128)
    tailp = jnp.concatenate(
        [tail, jnp.zeros((EPAD - NEDGE,), _i32)]).reshape(-1, 128)
    typep = jnp.concatenate(
        [etype, jnp.ones((EPAD - NEDGE,), _i32)]).reshape(-1, 128)
    rowp = jnp.concatenate(
        [row, jnp.full((NNZPAD - NNZ,), NU, _i32)]).reshape(-1, 128)
    colp = jnp.concatenate(
        [col, jnp.zeros((NNZPAD - NNZ,), _i32)]).reshape(-1, 128)
    valp = jnp.concatenate(
        [val, jnp.zeros((NNZPAD - NNZ,), _f32)]).reshape(-1, 128)
    r2 = _perm_halves(r_emb)

    cntp = _sc_cnt(headp)
    intent_upd, cor2 = _tc_intent(intent_emb, intent_emb.T, r_emb)

    def hop_body(carry, _):
        e, u, eres, ures = carry
        esum, usum = _sc_hop(e.astype(_bf16).reshape(2 * NE, HALF), r2,
                             headp, tailp, typep, rowp, colp, valp)
        enew, eres2 = _tc_entity(esum, cntp, eres)
        unew, ures2 = _tc_user(usum, u, intent_upd, ures)
        return (enew, unew, eres2, ures2), None

    (_, _, eres, ures), _ = lax.scan(
        hop_body, (entity_emb, user_emb, entity_emb, user_emb), None, length=2)
    return eres, ures, cor2.reshape(())


# speedup vs baseline: 1.4202x; 1.0280x over previous
import functools

import jax
import jax.numpy as jnp
from jax import lax
from jax.experimental import pallas as pl
from jax.experimental.pallas import tpu as pltpu
from jax.experimental.pallas import tpu_sc as plsc

NE = 50000
NU = 20000
NI = 5
EMB = 64
HALF = 32
NREL = 23
NEDGE = 800000
NNZ = 1000000

CHUNK = 384
GRP = 3
EPAD = 811008
NNZPAD = 1007616
ECHUNKS_PER_TILE = 132
UCHUNKS_PER_TILE = 164
NE_ACC = 50008
CPROWS = 500

_f32 = jnp.float32
_bf16 = jnp.bfloat16
_i32 = jnp.int32

_MESH = dict(core_axis_name="c", subcore_axis_name="s")


def _perm_halves(x):
    n = x.shape[0]
    return x.reshape(n, 2, 16, 2).swapaxes(2, 3).reshape(2 * n, 32)


def _pack_bf16(pa, pb):
    return plsc.pack(pa, pb, format=plsc.PackFormat.INTERLEAVED)


_SC_PARAMS = pltpu.CompilerParams(
    use_tc_tiling_on_sc=False, needs_layout_passes=False)


def _sc_cnt(headp):
    @functools.partial(
        pl.kernel,
        out_type=jax.ShapeDtypeStruct((2, NE, 16), _f32),
        mesh=plsc.VectorSubcoreMesh(**_MESH),
        compiler_params=_SC_PARAMS,
        scratch_types=[
            pltpu.VMEM((CHUNK, 16), _f32),
            pltpu.VMEM((CPROWS, 16), _f32),
            pltpu.VMEM((GRP, 128), _i32),
            pltpu.VMEM_SHARED((NE_ACC, 16), _f32),
            pltpu.SemaphoreType.DMA,
        ],
    )
    def k(headp_h, out_h, onesb, zb, raw0, cacc, sem):
        c = lax.axis_index("c")
        s = lax.axis_index("s")
        onev = jnp.ones((16,), _f32)
        zv = jnp.zeros((16,), _f32)

        def fill_body(i, _):
            onesb[i, pl.ds(0, 16)] = onev
            return 0

        lax.fori_loop(0, CHUNK, fill_body, 0, unroll=8)

        def z_body(i, _):
            zb[i, pl.ds(0, 16)] = zv
            return 0

        lax.fori_loop(0, CPROWS, z_body, 0, unroll=8)

        def zc_body(z, _):
            r = s + z * 16

            @pl.when(r < 100)
            def _():
                pltpu.sync_copy(zb, cacc.at[pl.ds(r * CPROWS, CPROWS), :])
            return 0

        lax.fori_loop(0, 7, zc_body, 0)
        plsc.subcore_barrier()

        def chunk_body(i, _):
            j = (c * 16 + s) * 66 + i
            pltpu.sync_copy(headp_h.at[pl.ds(j * GRP, GRP), :], raw0)
            for g in range(GRP):
                pltpu.sync_copy(
                    onesb.at[pl.ds(g * 128, 128), :],
                    cacc.at[raw0.at[g]],
                    add=True,
                )
            return 0

        lax.fori_loop(0, 66, chunk_body, 0)
        plsc.subcore_barrier()

        def cp_body(z, _):
            r = s + z * 16

            @pl.when(r < 100)
            def _():
                pltpu.sync_copy(cacc.at[pl.ds(r * CPROWS, CPROWS), :], zb)
                pltpu.sync_copy(zb, out_h.at[c, pl.ds(r * CPROWS, CPROWS), :])
            return 0

        lax.fori_loop(0, 7, cp_body, 0)

    return k(headp)


def _sc_hop(ent2, r2, headp, tailp, typep, rowp, colp, valp):
    @functools.partial(
        pl.kernel,
        out_type=[
            jax.ShapeDtypeStruct((2, NE, HALF), _bf16),
            jax.ShapeDtypeStruct((2, NU, HALF), _bf16),
        ],
        mesh=plsc.VectorSubcoreMesh(**_MESH),
        compiler_params=_SC_PARAMS,
        scratch_types=[
            pltpu.VMEM((CHUNK, HALF), _bf16),
            pltpu.VMEM((CHUNK, HALF), _bf16),
            pltpu.VMEM((2 * NREL, HALF), _f32),
            pltpu.VMEM((CHUNK, HALF), _bf16),
            pltpu.VMEM((CHUNK, HALF), _bf16),
            pltpu.VMEM((GRP, 128), _i32),
            pltpu.VMEM((GRP, 128), _i32),
            pltpu.VMEM((GRP, 128), _i32),
            pltpu.VMEM((GRP, 128), _i32),
            pltpu.VMEM((GRP, 128), _i32),
            pltpu.VMEM((GRP, 128), _i32),
            pltpu.VMEM((GRP, 128), _f32),
            pltpu.VMEM((GRP, 128), _f32),
            pltpu.VMEM((GRP, 128), _i32),
            pltpu.VMEM((GRP, 128), _i32),
            pltpu.VMEM((GRP, 128), _i32),
            pltpu.VMEM((CPROWS, HALF), _bf16),
            pltpu.VMEM_SHARED((NE_ACC, HALF), _bf16),
            pltpu.SemaphoreType.DMA,
            pltpu.SemaphoreType.DMA,
            pltpu.SemaphoreType.DMA,
            pltpu.SemaphoreType.DMA,
            pltpu.SemaphoreType.DMA,
        ],
    )
    def k(ent2_h, r2_h, headp_h, tailp_h, typep_h, rowp_h, colp_h, valp_h,
          esum_h, usum_h,
          tA0, tA1, rtmp, tP0, tP1, gidx0, gidx1, relg0, relg1,
          sidx0, sidx1, valb0, valb1, raw0, raw1, raw2, zb, acc,
          sem, gsa, gsb, ssa, ssb):
        c = lax.axis_index("c")
        s = lax.axis_index("s")
        zv = jnp.zeros((32,), _bf16)
        A = dict(tA=tA0, tP=tP0, gidx=gidx0, relg=relg0, sidx=sidx0,
                 valb=valb0, gs=gsa, ss=ssa)
        B = dict(tA=tA1, tP=tP1, gidx=gidx1, relg=relg1, sidx=sidx1,
                 valb=valb1, gs=gsb, ss=ssb)
        pltpu.sync_copy(r2_h, rtmp)

        def z_body(i, _):
            zb[i, :] = zv
            return 0

        def zero_rows(nchunks, zmax):
            def zc_body(z, _):
                r = s + z * 16

                @pl.when(r < nchunks)
                def _():
                    pltpu.sync_copy(zb, acc.at[pl.ds(r * CPROWS, CPROWS), :])
                return 0

            lax.fori_loop(0, zmax, zc_body, 0)

        def copy_out(nchunks, zmax, out_ref):
            def cp_body(z, _):
                r = s + z * 16

                @pl.when(r < nchunks)
                def _():
                    pltpu.sync_copy(acc.at[pl.ds(r * CPROWS, CPROWS), :], zb)
                    pltpu.sync_copy(
                        zb, out_ref.at[c, pl.ds(r * CPROWS, CPROWS), :])
                return 0

            lax.fori_loop(0, zmax, cp_body, 0)

        lax.fori_loop(0, CPROWS, z_body, 0, unroll=8)
        zero_rows(100, 7)
        plsc.subcore_barrier()

        ebase = s * ECHUNKS_PER_TILE

        def eload_raws(n):
            base = (ebase + n) * GRP
            dl = [pltpu.async_copy(headp_h.at[pl.ds(base, GRP), :], raw0, sem),
                  pltpu.async_copy(tailp_h.at[pl.ds(base, GRP), :], raw1, sem),
                  pltpu.async_copy(typep_h.at[pl.ds(base, GRP), :], raw2, sem)]
            for d in dl:
                d.wait()

        def eidx(S):
            def idxb(q, _):
                g = q // 8
                off = (q % 8) * 16
                t = raw1[g, pl.ds(off, 16)]
                S["gidx"][g, pl.ds(off, 16)] = t + t + c
                ty = raw2[g, pl.ds(off, 16)] + 22
                m = ty - jnp.where(ty >= NREL, NREL, 0).astype(_i32)
                S["relg"][g, pl.ds(off, 16)] = m + m + c
                S["sidx"][g, pl.ds(off, 16)] = raw0[g, pl.ds(off, 16)]
                return 0

            lax.fori_loop(0, 24, idxb, 0, unroll=4)

        def efire_gathers(S):
            for g in range(GRP):
                pltpu.async_copy(ent2_h.at[S["gidx"].at[g]],
                                 S["tA"].at[pl.ds(g * 128, 128), :], S["gs"])

        def ewait_gathers(S):
            pltpu.make_async_copy(
                ent2_h.at[pl.ds(0, CHUNK)], S["tA"], S["gs"]).wait()

        def edrain_scatters(S):
            pltpu.make_async_copy(
                esum_h.at[c, pl.ds(0, CHUNK), :], S["tP"], S["ss"]).wait()

        dn0 = lax.GatherDimensionNumbers(
            offset_dims=(), collapsed_slice_dims=(0,), start_index_map=(0,))
        iota16 = jnp.arange(16, dtype=_i32)
        iota16p = iota16 + 16

        def emul(S):
            def mulb(q, _):
                g = q // 8
                off = (q % 8) * 16
                rr = S["relg"][g, pl.ds(off, 16)]
                for lane in range(16):
                    li = jnp.full((16, 1), lane, _i32)
                    rsp = lax.gather(
                        rr, li, dn0, slice_sizes=(1,),
                        mode=lax.GatherScatterMode.PROMISE_IN_BOUNDS)
                    va = plsc.load_gather(rtmp, [rsp, iota16])
                    vb = plsc.load_gather(rtmp, [rsp, iota16p])
                    k2 = q * 16 + lane
                    w = plsc.bitcast(S["tA"][k2, :], _i32)
                    ea = plsc.bitcast(lax.shift_left(w, 16), _f32)
                    eb = plsc.bitcast(w & jnp.int32(-65536), _f32)
                    S["tP"][k2, :] = _pack_bf16(ea * va, eb * vb)
                return 0

            lax.fori_loop(0, 24, mulb, 0)

        def efire_scatters(S):
            for g in range(GRP):
                pltpu.async_copy(S["tP"].at[pl.ds(g * 128, 128), :],
                                 acc.at[S["sidx"].at[g]], S["ss"], add=True)

        eload_raws(0)
        eidx(A)
        efire_gathers(A)

        NP = ECHUNKS_PER_TILE // 2

        def epair(i, _):
            @pl.when(i > 0)
            def _():
                edrain_scatters(B)
            eload_raws(2 * i + 1)
            eidx(B)
            efire_gathers(B)
            ewait_gathers(A)
            emul(A)
            efire_scatters(A)
            @pl.when(i < NP - 1)
            def _():
                edrain_scatters(A)
                eload_raws(2 * i + 2)
                eidx(A)
                efire_gathers(A)
            ewait_gathers(B)
            emul(B)
            efire_scatters(B)
            return 0

        lax.fori_loop(0, NP, epair, 0)
        edrain_scatters(A)
        edrain_scatters(B)
        plsc.subcore_barrier()
        copy_out(100, 7, esum_h)
        plsc.subcore_barrier()

        lax.fori_loop(0, CPROWS, z_body, 0, unroll=8)
        zero_rows(40, 3)
        plsc.subcore_barrier()

        ubase = s * UCHUNKS_PER_TILE
        dn = lax.GatherDimensionNumbers(
            offset_dims=(), collapsed_slice_dims=(0,), start_index_map=(0,))

        def uload_raws(n, S):
            base = (ubase + n) * GRP
            dl = [pltpu.async_copy(rowp_h.at[pl.ds(base, GRP), :], raw0, sem),
                  pltpu.async_copy(colp_h.at[pl.ds(base, GRP), :], raw1, sem),
                  pltpu.async_copy(valp_h.at[pl.ds(base, GRP), :],
                                   S["valb"], sem)]
            for d in dl:
                d.wait()

        def uidx(S):
            def idxb(q, _):
                g = q // 8
                off = (q % 8) * 16
                t = raw1[g, pl.ds(off, 16)]
                S["gidx"][g, pl.ds(off, 16)] = t + t + c
                S["sidx"][g, pl.ds(off, 16)] = raw0[g, pl.ds(off, 16)]
                return 0

            lax.fori_loop(0, 24, idxb, 0, unroll=4)

        def ufire_gathers(S):
            for g in range(GRP):
                pltpu.async_copy(ent2_h.at[S["gidx"].at[g]],
                                 S["tA"].at[pl.ds(g * 128, 128), :], S["gs"])

        def uwait_gathers(S):
            pltpu.make_async_copy(
                ent2_h.at[pl.ds(0, CHUNK)], S["tA"], S["gs"]).wait()

        def umul(S):
            def vmulb(q, _):
                g = q // 8
                off = (q % 8) * 16
                vals16 = S["valb"][g, pl.ds(off, 16)]
                for lane in range(16):
                    li = jnp.full((16, 1), lane, _i32)
                    vs = lax.gather(
                        vals16, li, dn, slice_sizes=(1,),
                        mode=lax.GatherScatterMode.PROMISE_IN_BOUNDS)
                    k2 = q * 16 + lane
                    w = plsc.bitcast(S["tA"][k2, :], _i32)
                    ea = plsc.bitcast(lax.shift_left(w, 16), _f32)
                    eb = plsc.bitcast(w & jnp.int32(-65536), _f32)
                    S["tP"][k2, :] = _pack_bf16(ea * vs, eb * vs)
                return 0

            lax.fori_loop(0, 24, vmulb, 0)

        uload_raws(0, A)
        uidx(A)
        ufire_gathers(A)

        NUP = UCHUNKS_PER_TILE // 2

        def upair(i, _):
            @pl.when(i > 0)
            def _():
                edrain_scatters(B)
            uload_raws(2 * i + 1, B)
            uidx(B)
            ufire_gathers(B)
            uwait_gathers(A)
            umul(A)
            efire_scatters(A)

            @pl.when(i < NUP - 1)
            def _():
                edrain_scatters(A)
                uload_raws(2 * i + 2, A)
                uidx(A)
                ufire_gathers(A)
            uwait_gathers(B)
            umul(B)
            efire_scatters(B)
            return 0

        lax.fori_loop(0, NUP, upair, 0)
        edrain_scatters(A)
        edrain_scatters(B)
        plsc.subcore_barrier()
        copy_out(40, 3, usum_h)

    return k(ent2, r2, headp, tailp, typep, rowp, colp, valp)


def _dcorr_centered(tr, tc):
    outer = tc * tr
    a = jnp.sqrt(jnp.maximum(tc * tc - 2.0 * outer + tr * tr, 0.0) + 1e-8)
    return (a - jnp.mean(a, axis=0, keepdims=True)
            - jnp.mean(a, axis=1, keepdims=True) + jnp.mean(a))


def _dcorr(t1r, t1c, t2r, t2c):
    n2 = float(EMB * EMB)
    A = _dcorr_centered(t1r, t1c)
    B = _dcorr_centered(t2r, t2c)
    dab = jnp.sqrt(jnp.maximum(jnp.sum(A * B) / n2, 0.0) + 1e-8)
    daa = jnp.sqrt(jnp.maximum(jnp.sum(A * A) / n2, 0.0) + 1e-8)
    dbb = jnp.sqrt(jnp.maximum(jnp.sum(B * B) / n2, 0.0) + 1e-8)
    return dab / jnp.sqrt(daa * dbb + 1e-8)


def _tc_intent_body(intent_ref, intentT_ref, r_ref, upd_ref, cor_ref):
    intent = intent_ref[...]
    intentT = intentT_ref[...]
    r = r_ref[...]

    def part(vec_row, emb):
        sc = jnp.sum(vec_row * emb, axis=1, keepdims=True)
        att = jax.nn.softmax(sc, axis=0)
        return jnp.mean(att * emb, axis=0, keepdims=True)

    parts = [part(intent[0:1], r),
             part(intent[1:2], r[0:6]),
             part(intent[2:3], r[6:12]),
             part(intent[3:4], r[12:18]),
             part(intent[4:5], r[18:23])]
    all_intent = jnp.concatenate(parts, axis=0)
    upd_ref[...] = (all_intent + intent) / 2.0
    cor = jnp.float32(0.0)
    for i in range(NI):
        for j in range(i + 1, NI):
            cor = cor + _dcorr(intent[i:i + 1, :], intentT[:, i:i + 1],
                               intent[j:j + 1, :], intentT[:, j:j + 1])
    cor_ref[...] = cor.reshape(1, 1)


def _tc_intent(intent_emb, intent_embT, r_emb):
    return pl.pallas_call(
        _tc_intent_body,
        out_shape=[jax.ShapeDtypeStruct((NI, EMB), _f32),
                   jax.ShapeDtypeStruct((1, 1), _f32)],
    )(intent_emb, intent_embT, r_emb)


def _tc_ent_body(esum_ref, cnt_ref, res_ref, e_ref, rout_ref):
    es = jnp.concatenate([esum_ref[0], esum_ref[1]], axis=1).astype(_f32)
    cnt = cnt_ref[0, :, 0:1] + cnt_ref[1, :, 0:1]
    x = es / jnp.maximum(cnt, 1.0)
    nrm = jnp.sqrt(jnp.sum(x * x, axis=1, keepdims=True))
    e = x / jnp.maximum(nrm, 1e-12)
    e_ref[...] = e
    rout_ref[...] = res_ref[...] + e


def _tc_entity(esum, cntp, res_in):
    blk = 2000
    return pl.pallas_call(
        _tc_ent_body,
        grid=(NE // blk,),
        in_specs=[
            pl.BlockSpec((2, blk, HALF), lambda i: (0, i, 0)),
            pl.BlockSpec((2, blk, 16), lambda i: (0, i, 0)),
            pl.BlockSpec((blk, EMB), lambda i: (i, 0)),
        ],
        out_specs=[
            pl.BlockSpec((blk, EMB), lambda i: (i, 0)),
            pl.BlockSpec((blk, EMB), lambda i: (i, 0)),
        ],
        out_shape=[jax.ShapeDtypeStruct((NE, EMB), _f32),
                   jax.ShapeDtypeStruct((NE, EMB), _f32)],
    )(esum, cntp, res_in)


def _tc_user_body(usum_ref, uprev_ref, upd_ref, res_ref, u_ref, rout_ref):
    us = jnp.concatenate([usum_ref[0], usum_ref[1]], axis=1).astype(_f32)
    up = upd_ref[...]
    score_ = lax.dot_general(uprev_ref[...], up, (((1,), (1,)), ((), ())),
                             preferred_element_type=_f32)
    score = jax.nn.softmax(score_, axis=1)
    f = lax.dot_general(score, up, (((1,), (0,)), ((), ())),
                        preferred_element_type=_f32)
    u = us * (1.0 + f)
    nrm = jnp.sqrt(jnp.sum(u * u, axis=1, keepdims=True))
    un = u / jnp.maximum(nrm, 1e-12)
    u_ref[...] = un
    rout_ref[...] = res_ref[...] + un


def _tc_user(usum, u_prev, intent_upd, res_in):
    blk = 2000
    return pl.pallas_call(
        _tc_user_body,
        grid=(NU // blk,),
        in_specs=[
            pl.BlockSpec((2, blk, HALF), lambda i: (0, i, 0)),
            pl.BlockSpec((blk, EMB), lambda i: (i, 0)),
            pl.BlockSpec((NI, EMB), lambda i: (0, 0)),
            pl.BlockSpec((blk, EMB), lambda i: (i, 0)),
        ],
        out_specs=[
            pl.BlockSpec((blk, EMB), lambda i: (i, 0)),
            pl.BlockSpec((blk, EMB), lambda i: (i, 0)),
        ],
        out_shape=[jax.ShapeDtypeStruct((NU, EMB), _f32),
                   jax.ShapeDtypeStruct((NU, EMB), _f32)],
    )(usum, u_prev, intent_upd, res_in)


def kernel(entity_emb, user_emb, intent_emb, edge_index, edge_type,
           interact_row, interact_col, interact_val, r_emb):
    head = edge_index[0].astype(_i32)
    tail = edge_index[1].astype(_i32)
    etype = edge_type.astype(_i32)
    row = interact_row.astype(_i32)
    col = interact_col.astype(_i32)
    val = interact_val.astype(_f32)

    headp = jnp.concatenate(
        [head, jnp.full((EPAD - NEDGE,), NE, _i32)]).reshape(-1, 128)
    tailp = jnp.concatenate(
        [tail, jnp.zeros((EPAD - NEDGE,), _i32)]).reshape(-1, 128)
    typep = jnp.concatenate(
        [etype, jnp.ones((EPAD - NEDGE,), _i32)]).reshape(-1, 128)
    rowp = jnp.concatenate(
        [row, jnp.full((NNZPAD - NNZ,), NU, _i32)]).reshape(-1, 128)
    colp = jnp.concatenate(
        [col, jnp.zeros((NNZPAD - NNZ,), _i32)]).reshape(-1, 128)
    valp = jnp.concatenate(
        [val, jnp.zeros((NNZPAD - NNZ,), _f32)]).reshape(-1, 128)
    r2 = _perm_halves(r_emb)

    cntp = _sc_cnt(headp)
    intent_upd, cor2 = _tc_intent(intent_emb, intent_emb.T, r_emb)

    def hop_body(carry, _):
        e, u, eres, ures = carry
        esum, usum = _sc_hop(e.astype(_bf16).reshape(2 * NE, HALF), r2,
                             headp, tailp, typep, rowp, colp, valp)
        enew, eres2 = _tc_entity(esum, cntp, eres)
        unew, ures2 = _tc_user(usum, u, intent_upd, ures)
        return (enew, unew, eres2, ures2), None

    (_, _, eres, ures), _ = lax.scan(
        hop_body, (entity_emb, user_emb, entity_emb, user_emb), None, length=2)
    return eres, ures, cor2.reshape(())
